# baseline (speedup 1.0000x reference)
import functools

import jax
import jax.numpy as jnp
from jax import lax
from jax.experimental import pallas as pl
from jax.experimental.pallas import tpu as pltpu
from jax.experimental.pallas import tpu_sc as plsc

B = 16384
D = 16
NC = 2
NS = 16
NW = NC * NS
ROWS = B // NW
CH = 128
NCH = ROWS // CH
LANE = 128
RPB = LANE // D
GRP = ROWS // D


def _irt_body(theta_hbm, alpha_hbm, betab_hbm, sblk_hbm, qblk_hbm,
              soff_hbm, qoff_hbm, boff_hbm, out_hbm,
              sblk_v, qblk_v, bblk_v, soff_v, qoff_v, boff_v,
              tb, ab, bb, out_v, sem0, sem1):
    wid = lax.axis_index("s") * NC + lax.axis_index("c")
    base = wid * ROWS

    pltpu.sync_copy(sblk_hbm.at[wid], sblk_v)
    pltpu.sync_copy(qblk_hbm.at[wid, 0], qblk_v)
    pltpu.sync_copy(qblk_hbm.at[wid, 1], bblk_v)
    pltpu.sync_copy(soff_hbm.at[wid], soff_v)
    pltpu.sync_copy(qoff_hbm.at[wid], qoff_v)
    pltpu.sync_copy(boff_hbm.at[wid], boff_v)

    def fire(j):
        b = j % 2
        s = sem0 if b == 0 else sem1
        return [
            pltpu.async_copy(theta_hbm.at[sblk_v.at[j]], tb.at[b], s),
            pltpu.async_copy(alpha_hbm.at[qblk_v.at[j]], ab.at[b], s),
            pltpu.async_copy(betab_hbm.at[bblk_v.at[j]], bb.at[b], s),
        ]

    def compute_chunk(j):
        b = j % 2

        def group(g, carry):
            rowbase = g * D
            absrow = j * CH + rowbase
            soff = soff_v[pl.ds(absrow, D)]
            qoff = qoff_v[pl.ds(absrow, D)]
            boff = boff_v[pl.ds(absrow, D)]
            rows = lax.broadcasted_iota(jnp.int32, (D,), 0) + rowbase
            bvals = plsc.load_gather(bb.at[b], [rows, boff])
            orow = j * (CH // RPB) + g * (D // RPB)
            for k in range(D):
                r = rowbase + k
                t = tb[b, r, pl.ds(soff[k], D)]
                a = ab[b, r, pl.ds(qoff[k], D)]
                x = 1.702 * (a * (t - bvals[k]))
                out_v[orow + k // RPB, pl.ds((k % RPB) * D, D)] = (
                    1.0 / (1.0 + jnp.exp(-x)))
            return carry

        lax.fori_loop(0, CH // D, group, 0)

    pend = fire(0)
    for j in range(NCH):
        nxt = fire(j + 1) if j + 1 < NCH else None
        for cp in pend:
            cp.wait()
        compute_chunk(j)
        pend = nxt

    pltpu.sync_copy(out_v, out_hbm.at[pl.ds(wid * (ROWS * D // LANE),
                                            ROWS * D // LANE)])


@jax.jit
def kernel(theta_table, alpha_table, beta_table, student_ids, question_ids):
    sid = student_ids.astype(jnp.int32)
    qid = question_ids.astype(jnp.int32)

    theta128 = theta_table.reshape(-1, LANE)
    alpha128 = alpha_table.reshape(-1, LANE)
    nbeta = beta_table.shape[0]
    nbblk = (nbeta + LANE - 1) // LANE
    beta128 = jnp.pad(beta_table.reshape(-1),
                      (0, nbblk * LANE - nbeta)).reshape(nbblk, LANE)

    sblk = (sid // RPB).reshape(NW, NCH, CH)
    soff = ((sid % RPB) * D).reshape(NW, ROWS)
    qblk2 = jnp.stack([(qid // RPB).reshape(NW, NCH, CH),
                       (qid // LANE).reshape(NW, NCH, CH)], axis=1)
    qoff = ((qid % RPB) * D).reshape(NW, ROWS)
    boff = (qid % LANE).reshape(NW, ROWS)

    mesh = plsc.VectorSubcoreMesh(core_axis_name="c", subcore_axis_name="s")
    run = pl.kernel(
        _irt_body,
        mesh=mesh,
        out_type=jax.ShapeDtypeStruct((B * D // LANE, LANE), jnp.float32),
        compiler_params=pltpu.CompilerParams(needs_layout_passes=False),
        scratch_types=[
            pltpu.VMEM((NCH, CH), jnp.int32),
            pltpu.VMEM((NCH, CH), jnp.int32),
            pltpu.VMEM((NCH, CH), jnp.int32),
            pltpu.VMEM((ROWS,), jnp.int32),
            pltpu.VMEM((ROWS,), jnp.int32),
            pltpu.VMEM((ROWS,), jnp.int32),
            pltpu.VMEM((2, CH, LANE), jnp.float32),
            pltpu.VMEM((2, CH, LANE), jnp.float32),
            pltpu.VMEM((2, CH, LANE), jnp.float32),
            pltpu.VMEM((ROWS * D // LANE, LANE), jnp.float32),
            pltpu.SemaphoreType.DMA,
            pltpu.SemaphoreType.DMA,
        ],
    )
    out = run(theta128, alpha128, beta128, sblk, qblk2, soff, qoff, boff)
    return out.reshape(B, D)


# speedup vs baseline: 3.6028x; 3.6028x over previous
import jax
import jax.numpy as jnp
from jax import lax
from jax.experimental import pallas as pl
from jax.experimental.pallas import tpu as pltpu
from jax.experimental.pallas import tpu_sc as plsc

B = 16384
D = 16
NC = 2
NS = 16
NW = NC * NS
ROWS = B // NW
W = 32
NWAVE = ROWS // W
LANE = 128
GRP = 16
RPB = LANE // D


def _irt_body(theta_hbm, alpha_hbm, betab_hbm,
              sc_hbm, sl_hbm, ab_hbm, ao_hbm, bb_hbm, bl_hbm, out_hbm,
              sc_v, sl_v, ab_v, ao_v, bb_v, bl_v,
              tbuf, abuf, bbuf, out_v, sem):
    wid = lax.axis_index("s") * NC + lax.axis_index("c")

    pltpu.sync_copy(sc_hbm.at[wid], sc_v)
    pltpu.sync_copy(sl_hbm.at[wid], sl_v)
    pltpu.sync_copy(ab_hbm.at[wid], ab_v)
    pltpu.sync_copy(ao_hbm.at[wid], ao_v)
    pltpu.sync_copy(bb_hbm.at[wid], bb_v)
    pltpu.sync_copy(bl_hbm.at[wid], bl_v)

    iota = lax.broadcasted_iota(jnp.int32, (GRP,), 0)

    def wave(w, carry):
        base = w * W
        copies = [
            pltpu.async_copy(alpha_hbm.at[ab_v.at[pl.ds(base, W)]],
                             abuf, sem),
            pltpu.async_copy(betab_hbm.at[bb_v.at[pl.ds(base, W)]],
                             bbuf, sem),
        ]
        for g in range(W // GRP):
            scc = sc_v[pl.ds(base + g * GRP, GRP)]
            for k in range(GRP):
                e = g * GRP + k
                copies.append(pltpu.async_copy(
                    theta_hbm.at[:, pl.ds(scc[k] * LANE, LANE)],
                    tbuf.at[:, pl.ds(e * LANE, LANE)], sem))
        for cp in copies:
            cp.wait()

        for g in range(W // GRP):
            r0 = base + g * GRP
            sl = sl_v[pl.ds(r0, GRP)]
            ao = ao_v[pl.ds(r0, GRP)]
            bl = bl_v[pl.ds(r0, GRP)]
            erow = iota + g * GRP
            tlane = erow * LANE + sl
            bvals = plsc.load_gather(bbuf, [erow, bl])
            for d in range(D):
                dvec = jnp.full((GRP,), d, jnp.int32)
                t = plsc.load_gather(tbuf, [dvec, tlane])
                a = plsc.load_gather(abuf, [erow, ao + d])
                x = 1.702 * (a * (t - bvals))
                out_v[d, pl.ds(r0, GRP)] = 1.0 / (1.0 + jnp.exp(-x))
        return carry

    lax.fori_loop(0, NWAVE, wave, 0)

    pltpu.sync_copy(out_v, out_hbm.at[:, pl.ds(wid * ROWS, ROWS)])


@jax.jit
def kernel(theta_table, alpha_table, beta_table, student_ids, question_ids):
    sid = student_ids.astype(jnp.int32)
    qid = question_ids.astype(jnp.int32)

    theta_t = theta_table.T
    alpha128 = alpha_table.reshape(-1, LANE)
    nbeta = beta_table.shape[0]
    nbblk = (nbeta + LANE - 1) // LANE
    beta128 = jnp.pad(beta_table.reshape(-1),
                      (0, nbblk * LANE - nbeta)).reshape(nbblk, LANE)

    sc = (sid // LANE).reshape(NW, ROWS)
    sl = (sid % LANE).reshape(NW, ROWS)
    ab = (qid // RPB).reshape(NW, ROWS)
    ao = ((qid % RPB) * D).reshape(NW, ROWS)
    bb = (qid // LANE).reshape(NW, ROWS)
    bl = (qid % LANE).reshape(NW, ROWS)

    mesh = plsc.VectorSubcoreMesh(core_axis_name="c", subcore_axis_name="s")
    run = pl.kernel(
        _irt_body,
        mesh=mesh,
        out_type=jax.ShapeDtypeStruct((D, B), jnp.float32),
        compiler_params=pltpu.CompilerParams(needs_layout_passes=False),
        scratch_types=[
            pltpu.VMEM((ROWS,), jnp.int32),
            pltpu.VMEM((ROWS,), jnp.int32),
            pltpu.VMEM((ROWS,), jnp.int32),
            pltpu.VMEM((ROWS,), jnp.int32),
            pltpu.VMEM((ROWS,), jnp.int32),
            pltpu.VMEM((ROWS,), jnp.int32),
            pltpu.VMEM((D, W * LANE), jnp.float32),
            pltpu.VMEM((W, LANE), jnp.float32),
            pltpu.VMEM((W, LANE), jnp.float32),
            pltpu.VMEM((D, ROWS), jnp.float32),
            pltpu.SemaphoreType.DMA,
        ],
    )
    out_t = run(theta_t, alpha128, beta128, sc, sl, ab, ao, bb, bl)
    return out_t.T


# speedup vs baseline: 3.8735x; 1.0752x over previous
import jax
import jax.numpy as jnp
from jax import lax
from jax.experimental import pallas as pl
from jax.experimental.pallas import tpu as pltpu
from jax.experimental.pallas import tpu_sc as plsc

B = 16384
D = 16
NC = 2
NS = 16
NW = NC * NS
ROWS = B // NW
W = 16
NWAVE = ROWS // W
LANE = 128
GRP = 16
RPB = LANE // D


def _irt_body(theta_hbm, alpha_hbm, betab_hbm,
              sc_hbm, sl_hbm, ab_hbm, ao_hbm, bb_hbm, bl_hbm, out_hbm,
              sc_v, sl_v, ab_v, ao_v, bb_v, bl_v,
              tbuf0, tbuf1, abuf0, abuf1, bbuf0, bbuf1, out_v,
              sem0, sem1):
    wid = lax.axis_index("s") * NC + lax.axis_index("c")

    pltpu.sync_copy(sc_hbm.at[wid], sc_v)
    pltpu.sync_copy(sl_hbm.at[wid], sl_v)
    pltpu.sync_copy(ab_hbm.at[wid], ab_v)
    pltpu.sync_copy(ao_hbm.at[wid], ao_v)
    pltpu.sync_copy(bb_hbm.at[wid], bb_v)
    pltpu.sync_copy(bl_hbm.at[wid], bl_v)

    iota = lax.broadcasted_iota(jnp.int32, (GRP,), 0)

    def fire(w, tb, ab, bb, s):
        base = w * W
        pltpu.async_copy(alpha_hbm.at[ab_v.at[pl.ds(base, W)]], ab, s)
        pltpu.async_copy(betab_hbm.at[bb_v.at[pl.ds(base, W)]], bb, s)
        scc = sc_v[pl.ds(base, W)]
        for k in range(W):
            pltpu.async_copy(theta_hbm.at[:, pl.ds(scc[k] * LANE, LANE)],
                             tb.at[:, pl.ds(k * LANE, LANE)], s)

    def drain(tb, ab, bb, s):
        pltpu.make_async_copy(alpha_hbm.at[pl.ds(0, W)], ab, s).wait()
        pltpu.make_async_copy(betab_hbm.at[pl.ds(0, W)], bb, s).wait()
        for k in range(W):
            pltpu.make_async_copy(theta_hbm.at[:, pl.ds(0, LANE)],
                                  tb.at[:, pl.ds(k * LANE, LANE)], s).wait()

    def compute(w, tb, ab, bb):
        r0 = w * W
        sl = sl_v[pl.ds(r0, GRP)]
        ao = ao_v[pl.ds(r0, GRP)]
        bl = bl_v[pl.ds(r0, GRP)]
        tlane = iota * LANE + sl
        bvals = plsc.load_gather(bb, [iota, bl])
        for d in range(D):
            dvec = jnp.full((GRP,), d, jnp.int32)
            t = plsc.load_gather(tb, [dvec, tlane])
            a = plsc.load_gather(ab, [iota, ao + d])
            x = 1.702 * (a * (t - bvals))
            out_v[d, pl.ds(r0, GRP)] = 1.0 / (1.0 + jnp.exp(-x))

    fire(0, tbuf0, abuf0, bbuf0, sem0)

    def body(u, carry):
        w0 = u * 2
        fire(w0 + 1, tbuf1, abuf1, bbuf1, sem1)
        drain(tbuf0, abuf0, bbuf0, sem0)
        compute(w0, tbuf0, abuf0, bbuf0)

        @pl.when(w0 + 2 < NWAVE)
        def _():
            fire(w0 + 2, tbuf0, abuf0, bbuf0, sem0)

        drain(tbuf1, abuf1, bbuf1, sem1)
        compute(w0 + 1, tbuf1, abuf1, bbuf1)
        return carry

    lax.fori_loop(0, NWAVE // 2, body, 0)

    pltpu.sync_copy(out_v, out_hbm.at[:, pl.ds(wid * ROWS, ROWS)])


@jax.jit
def kernel(theta_table, alpha_table, beta_table, student_ids, question_ids):
    sid = student_ids.astype(jnp.int32)
    qid = question_ids.astype(jnp.int32)

    theta_t = theta_table.T
    alpha128 = alpha_table.reshape(-1, LANE)
    nbeta = beta_table.shape[0]
    nbblk = (nbeta + LANE - 1) // LANE
    beta128 = jnp.pad(beta_table.reshape(-1),
                      (0, nbblk * LANE - nbeta)).reshape(nbblk, LANE)

    sc = (sid // LANE).reshape(NW, ROWS)
    sl = (sid % LANE).reshape(NW, ROWS)
    ab = (qid // RPB).reshape(NW, ROWS)
    ao = ((qid % RPB) * D).reshape(NW, ROWS)
    bb = (qid // LANE).reshape(NW, ROWS)
    bl = (qid % LANE).reshape(NW, ROWS)

    mesh = plsc.VectorSubcoreMesh(core_axis_name="c", subcore_axis_name="s")
    run = pl.kernel(
        _irt_body,
        mesh=mesh,
        out_type=jax.ShapeDtypeStruct((D, B), jnp.float32),
        compiler_params=pltpu.CompilerParams(needs_layout_passes=False),
        scratch_types=[
            pltpu.VMEM((ROWS,), jnp.int32),
            pltpu.VMEM((ROWS,), jnp.int32),
            pltpu.VMEM((ROWS,), jnp.int32),
            pltpu.VMEM((ROWS,), jnp.int32),
            pltpu.VMEM((ROWS,), jnp.int32),
            pltpu.VMEM((ROWS,), jnp.int32),
            pltpu.VMEM((D, W * LANE), jnp.float32),
            pltpu.VMEM((D, W * LANE), jnp.float32),
            pltpu.VMEM((W, LANE), jnp.float32),
            pltpu.VMEM((W, LANE), jnp.float32),
            pltpu.VMEM((W, LANE), jnp.float32),
            pltpu.VMEM((W, LANE), jnp.float32),
            pltpu.VMEM((D, ROWS), jnp.float32),
            pltpu.SemaphoreType.DMA,
            pltpu.SemaphoreType.DMA,
        ],
    )
    out_t = run(theta_t, alpha128, beta128, sc, sl, ab, ao, bb, bl)
    return out_t.T


# speedup vs baseline: 3.9641x; 1.0234x over previous
import jax
import jax.numpy as jnp
from jax import lax
from jax.experimental import pallas as pl
from jax.experimental.pallas import tpu as pltpu
from jax.experimental.pallas import tpu_sc as plsc

B = 16384
D = 16
NC = 2
NS = 16
NW = NC * NS
ROWS = B // NW
W = 16
NWAVE = ROWS // W
LANE = 128
GRP = 16
RPB = LANE // D


def _irt_body(theta_hbm, alpha_hbm, betab_hbm,
              sc_hbm, sl_hbm, ab_hbm, ao_hbm, bb_hbm, bl_hbm, out_hbm,
              sc_v, sl_v, ab_v, ao_v, bb_v, bl_v,
              tbuf0, tbuf1, abuf0, abuf1, bbuf0, bbuf1, out_v,
              sem0, sem1):
    wid = lax.axis_index("s") * NC + lax.axis_index("c")

    staged = [pltpu.async_copy(src.at[wid], dst, sem0) for src, dst in
              ((sc_hbm, sc_v), (sl_hbm, sl_v), (ab_hbm, ab_v),
               (ao_hbm, ao_v), (bb_hbm, bb_v), (bl_hbm, bl_v))]
    for cp in staged:
        cp.wait()

    iota = lax.broadcasted_iota(jnp.int32, (GRP,), 0)

    def fire(w, tb, ab, bb, s):
        base = w * W
        pltpu.async_copy(alpha_hbm.at[ab_v.at[pl.ds(base, W)]], ab, s)
        pltpu.async_copy(betab_hbm.at[bb_v.at[pl.ds(base, W)]], bb, s)
        scc = sc_v[pl.ds(base, W)]
        for k in range(W):
            pltpu.async_copy(theta_hbm.at[:, pl.ds(scc[k] * LANE, LANE)],
                             tb.at[:, pl.ds(k * LANE, LANE)], s)

    def drain(tb, ab, bb, s):
        pltpu.make_async_copy(alpha_hbm.at[pl.ds(0, W)], ab, s).wait()
        pltpu.make_async_copy(betab_hbm.at[pl.ds(0, W)], bb, s).wait()
        for k in range(W):
            pltpu.make_async_copy(theta_hbm.at[:, pl.ds(0, LANE)],
                                  tb.at[:, pl.ds(k * LANE, LANE)], s).wait()

    def compute(w, tb, ab, bb):
        r0 = w * W
        sl = sl_v[pl.ds(r0, GRP)]
        ao = ao_v[pl.ds(r0, GRP)]
        bl = bl_v[pl.ds(r0, GRP)]
        tlane = iota * LANE + sl
        bvals = plsc.load_gather(bb, [iota, bl])
        for d in range(D):
            dvec = jnp.full((GRP,), d, jnp.int32)
            t = plsc.load_gather(tb, [dvec, tlane])
            a = plsc.load_gather(ab, [iota, ao + d])
            x = 1.702 * (a * (t - bvals))
            out_v[d, pl.ds(r0, GRP)] = 1.0 / (1.0 + jnp.exp(-x))

    fire(0, tbuf0, abuf0, bbuf0, sem0)

    def body(u, carry):
        w0 = u * 2
        fire(w0 + 1, tbuf1, abuf1, bbuf1, sem1)
        drain(tbuf0, abuf0, bbuf0, sem0)
        compute(w0, tbuf0, abuf0, bbuf0)

        @pl.when(w0 + 2 < NWAVE)
        def _():
            fire(w0 + 2, tbuf0, abuf0, bbuf0, sem0)

        drain(tbuf1, abuf1, bbuf1, sem1)
        compute(w0 + 1, tbuf1, abuf1, bbuf1)
        return carry

    lax.fori_loop(0, NWAVE // 2, body, 0)

    pltpu.sync_copy(out_v, out_hbm.at[:, pl.ds(wid * ROWS, ROWS)])


@jax.jit
def kernel(theta_table, alpha_table, beta_table, student_ids, question_ids):
    sid = student_ids.astype(jnp.int32)
    qid = question_ids.astype(jnp.int32)

    theta_t = theta_table.T
    alpha128 = alpha_table.reshape(-1, LANE)
    nbeta = beta_table.shape[0]
    nbblk = (nbeta + LANE - 1) // LANE
    beta128 = jnp.pad(beta_table.reshape(-1),
                      (0, nbblk * LANE - nbeta)).reshape(nbblk, LANE)

    sc = (sid // LANE).reshape(NW, ROWS)
    sl = (sid % LANE).reshape(NW, ROWS)
    ab = (qid // RPB).reshape(NW, ROWS)
    ao = ((qid % RPB) * D).reshape(NW, ROWS)
    bb = (qid // LANE).reshape(NW, ROWS)
    bl = (qid % LANE).reshape(NW, ROWS)

    mesh = plsc.VectorSubcoreMesh(core_axis_name="c", subcore_axis_name="s")
    run = pl.kernel(
        _irt_body,
        mesh=mesh,
        out_type=jax.ShapeDtypeStruct((D, B), jnp.float32),
        compiler_params=pltpu.CompilerParams(needs_layout_passes=False),
        scratch_types=[
            pltpu.VMEM((ROWS,), jnp.int32),
            pltpu.VMEM((ROWS,), jnp.int32),
            pltpu.VMEM((ROWS,), jnp.int32),
            pltpu.VMEM((ROWS,), jnp.int32),
            pltpu.VMEM((ROWS,), jnp.int32),
            pltpu.VMEM((ROWS,), jnp.int32),
            pltpu.VMEM((D, W * LANE), jnp.float32),
            pltpu.VMEM((D, W * LANE), jnp.float32),
            pltpu.VMEM((W, LANE), jnp.float32),
            pltpu.VMEM((W, LANE), jnp.float32),
            pltpu.VMEM((W, LANE), jnp.float32),
            pltpu.VMEM((W, LANE), jnp.float32),
            pltpu.VMEM((D, ROWS), jnp.float32),
            pltpu.SemaphoreType.DMA,
            pltpu.SemaphoreType.DMA,
        ],
    )
    out_t = run(theta_t, alpha128, beta128, sc, sl, ab, ao, bb, bl)
    return out_t.T


# speedup vs baseline: 3.9684x; 1.0011x over previous
import jax
import jax.numpy as jnp
from jax import lax
from jax.experimental import pallas as pl
from jax.experimental.pallas import tpu as pltpu
from jax.experimental.pallas import tpu_sc as plsc

B = 16384
D = 16
NC = 2
NS = 16
NW = NC * NS
ROWS = B // NW
W = 16
NWAVE = ROWS // W
LANE = 128
GRP = 16
RPB = LANE // D


def _irt_body(theta_hbm, alpha_hbm, betab_hbm, sid_hbm, qid_hbm, out_hbm,
              sid_v, qid_v,
              tbuf0, tbuf1, abuf0, abuf1, bbuf0, bbuf1, out_v,
              sem0, sem1):
    wid = lax.axis_index("s") * NC + lax.axis_index("c")

    for cp in [pltpu.async_copy(sid_hbm.at[wid], sid_v, sem0),
               pltpu.async_copy(qid_hbm.at[wid], qid_v, sem0)]:
        cp.wait()

    iota = lax.broadcasted_iota(jnp.int32, (GRP,), 0)

    def fire(w, tb, ab, bb, s):
        base = w * W
        qv = qid_v[pl.ds(base, W)]
        pltpu.async_copy(alpha_hbm.at[qv >> 3], ab, s)
        pltpu.async_copy(betab_hbm.at[qv >> 7], bb, s)
        slab = sid_v[pl.ds(base, W)] >> 7
        for k in range(W):
            pltpu.async_copy(theta_hbm.at[:, pl.ds(slab[k] * LANE, LANE)],
                             tb.at[:, pl.ds(k * LANE, LANE)], s)

    def drain(tb, ab, bb, s):
        pltpu.make_async_copy(alpha_hbm.at[pl.ds(0, W)], ab, s).wait()
        pltpu.make_async_copy(betab_hbm.at[pl.ds(0, W)], bb, s).wait()
        for k in range(W):
            pltpu.make_async_copy(theta_hbm.at[:, pl.ds(0, LANE)],
                                  tb.at[:, pl.ds(k * LANE, LANE)], s).wait()

    def compute(w, tb, ab, bb):
        r0 = w * W
        sv = sid_v[pl.ds(r0, GRP)]
        qv = qid_v[pl.ds(r0, GRP)]
        tlane = iota * LANE + (sv & (LANE - 1))
        ao = (qv & (RPB - 1)) * D
        bvals = plsc.load_gather(bb, [iota, qv & (LANE - 1)])
        for d in range(D):
            dvec = jnp.full((GRP,), d, jnp.int32)
            t = plsc.load_gather(tb, [dvec, tlane])
            a = plsc.load_gather(ab, [iota, ao + d])
            x = 1.702 * (a * (t - bvals))
            out_v[d, pl.ds(r0, GRP)] = 1.0 / (1.0 + jnp.exp(-x))

    fire(0, tbuf0, abuf0, bbuf0, sem0)

    def body(u, carry):
        w0 = u * 2
        fire(w0 + 1, tbuf1, abuf1, bbuf1, sem1)
        drain(tbuf0, abuf0, bbuf0, sem0)
        compute(w0, tbuf0, abuf0, bbuf0)

        @pl.when(w0 + 2 < NWAVE)
        def _():
            fire(w0 + 2, tbuf0, abuf0, bbuf0, sem0)

        drain(tbuf1, abuf1, bbuf1, sem1)
        compute(w0 + 1, tbuf1, abuf1, bbuf1)
        return carry

    lax.fori_loop(0, NWAVE // 2, body, 0)

    pltpu.sync_copy(out_v, out_hbm.at[:, pl.ds(wid * ROWS, ROWS)])


@jax.jit
def kernel(theta_table, alpha_table, beta_table, student_ids, question_ids):
    sid = student_ids.astype(jnp.int32).reshape(NW, ROWS)
    qid = question_ids.astype(jnp.int32).reshape(NW, ROWS)

    theta_t = theta_table.T
    alpha128 = alpha_table.reshape(-1, LANE)
    nbeta = beta_table.shape[0]
    nbblk = (nbeta + LANE - 1) // LANE
    beta128 = jnp.pad(beta_table.reshape(-1),
                      (0, nbblk * LANE - nbeta)).reshape(nbblk, LANE)

    mesh = plsc.VectorSubcoreMesh(core_axis_name="c", subcore_axis_name="s")
    run = pl.kernel(
        _irt_body,
        mesh=mesh,
        out_type=jax.ShapeDtypeStruct((D, B), jnp.float32),
        compiler_params=pltpu.CompilerParams(needs_layout_passes=False),
        scratch_types=[
            pltpu.VMEM((ROWS,), jnp.int32),
            pltpu.VMEM((ROWS,), jnp.int32),
            pltpu.VMEM((D, W * LANE), jnp.float32),
            pltpu.VMEM((D, W * LANE), jnp.float32),
            pltpu.VMEM((W, LANE), jnp.float32),
            pltpu.VMEM((W, LANE), jnp.float32),
            pltpu.VMEM((W, LANE), jnp.float32),
            pltpu.VMEM((W, LANE), jnp.float32),
            pltpu.VMEM((D, ROWS), jnp.float32),
            pltpu.SemaphoreType.DMA,
            pltpu.SemaphoreType.DMA,
        ],
    )
    out_t = run(theta_t, alpha128, beta128, sid, qid)
    return out_t.T
